# Initial kernel scaffold; baseline (speedup 1.0000x reference)
import jax
import jax.numpy as jnp
from jax.experimental import pallas as pl


def kernel(x, edge_index, W_msg, b_msg, W_att, b_att, W_ih, b_ih, W_hh, b_hh, gamma, beta):
    raise NotImplementedError("write your pallas kernel here")


# speedup vs baseline: 8.3691x; 8.3691x over previous
import functools

import jax
import jax.numpy as jnp
from jax import lax
from jax.experimental import pallas as pl
from jax.experimental.pallas import tpu as pltpu
from jax.experimental.pallas import tpu_sc as plsc

N = 10000
E = 320000
D = 128

NC = 2
NS = 16
NW = NC * NS
E_PW = E // NW
K = 80
NCHUNK = E_PW // K
NWIN = 5
CPW = NCHUNK // NWIN
STRIPE = 640
LAST_STRIPE = N - 15 * STRIPE

BN = 1000
GRID = N // BN


def _pre_body(x_ref, wm_ref, bm_ref, wa_ref, ba_ref, m_ref, sd_ref):
    xb = x_ref[...]
    wm = wm_ref[...]
    m_ref[...] = lax.dot_general(
        xb, wm, (((1,), (1,)), ((), ())),
        preferred_element_type=jnp.float32) + bm_ref[...]
    a_sd = wa_ref[...].reshape(2, D)
    sd = lax.dot_general(
        xb, a_sd, (((1,), (1,)), ((), ())),
        preferred_element_type=jnp.float32)
    is_s = (lax.broadcasted_iota(jnp.int32, (1, 2), 1) == 0).astype(jnp.float32)
    sd_ref[...] = sd + ba_ref[0, 0] * is_s


def _precompute(x, W_msg, b_msg, W_att, b_att):
    return pl.pallas_call(
        _pre_body,
        grid=(GRID,),
        in_specs=[
            pl.BlockSpec((BN, D), lambda i: (i, 0)),
            pl.BlockSpec((D, D), lambda i: (0, 0)),
            pl.BlockSpec((1, D), lambda i: (0, 0)),
            pl.BlockSpec((1, 2 * D), lambda i: (0, 0)),
            pl.BlockSpec((1, 1), lambda i: (0, 0)),
        ],
        out_specs=[
            pl.BlockSpec((BN, D), lambda i: (i, 0)),
            pl.BlockSpec((BN, 2), lambda i: (i, 0)),
        ],
        out_shape=[
            jax.ShapeDtypeStruct((N, D), jnp.float32),
            jax.ShapeDtypeStruct((N, 2), jnp.float32),
        ],
    )(x, W_msg, b_msg.reshape(1, D), W_att, b_att.reshape(1, 1))


def _sc_body(m_hbm, s_hbm, d_hbm, src_hbm, dst_hbm, out_hbm,
             acc_sh, s_v, d_v, srcv, dstv, rows, sem):
    cid = lax.axis_index("c")
    sid = lax.axis_index("s")
    wid = cid * NS + sid

    def zero_row(i, _):
        for j in range(D // 16):
            rows[i, pl.ds(j * 16, 16)] = jnp.zeros((16,), jnp.float32)
        return _
    lax.fori_loop(0, K, zero_row, None)
    nzero = jnp.where(sid == NS - 1, LAST_STRIPE // K, STRIPE // K)

    def zero_stripe(t, _):
        pltpu.sync_copy(rows, acc_sh.at[pl.ds(sid * STRIPE + t * K, K)])
        return _
    lax.fori_loop(0, nzero, zero_stripe, None)

    pltpu.sync_copy(s_hbm, s_v)
    pltpu.sync_copy(d_hbm, d_v)

    plsc.subcore_barrier()

    def window(w, _):
        pltpu.sync_copy(src_hbm.at[wid, w], srcv)
        pltpu.sync_copy(dst_hbm.at[wid, w], dstv)

        def chunk(c, _):
            pltpu.async_copy(m_hbm.at[srcv.at[c]], rows, sem).wait()

            def group(g, _):
                sg = srcv[c, pl.ds(g * 16, 16)]
                dg = dstv[c, pl.ds(g * 16, 16)]
                sv = plsc.load_gather(s_v, [sg])
                dv = plsc.load_gather(d_v, [dg])
                att = 1.0 / (1.0 + jnp.exp(-(sv + dv)))
                for e in range(16):
                    a16 = jnp.full((16,), att[e], jnp.float32)
                    r = g * 16 + e
                    for j in range(D // 16):
                        rows[r, pl.ds(j * 16, 16)] = (
                            rows[r, pl.ds(j * 16, 16)] * a16)
                return _
            lax.fori_loop(0, K // 16, group, None)

            pltpu.sync_copy(rows, acc_sh.at[dstv.at[c]], add=True)
            return _
        lax.fori_loop(0, CPW, chunk, None)
        return _
    lax.fori_loop(0, NWIN, window, None)

    plsc.subcore_barrier()

    @pl.when(sid < NS - 1)
    def _copy_main():
        pltpu.sync_copy(acc_sh.at[pl.ds(sid * STRIPE, STRIPE)],
                        out_hbm.at[cid, pl.ds(sid * STRIPE, STRIPE)])

    @pl.when(sid == NS - 1)
    def _copy_last():
        pltpu.sync_copy(acc_sh.at[pl.ds(sid * STRIPE, LAST_STRIPE)],
                        out_hbm.at[cid, pl.ds(sid * STRIPE, LAST_STRIPE)])


def _sc_aggregate(M, s, d, src3, dst3):
    mesh = plsc.VectorSubcoreMesh(core_axis_name="c", subcore_axis_name="s")
    k = pl.kernel(
        _sc_body,
        out_type=jax.ShapeDtypeStruct((NC, N, D), jnp.float32),
        mesh=mesh,
        compiler_params=pltpu.CompilerParams(needs_layout_passes=False),
        scratch_types=[
            pltpu.VMEM_SHARED((N, D), jnp.float32),
            pltpu.VMEM((N,), jnp.float32),
            pltpu.VMEM((N,), jnp.float32),
            pltpu.VMEM((CPW, K), jnp.int32),
            pltpu.VMEM((CPW, K), jnp.int32),
            pltpu.VMEM((K, D), jnp.float32),
            pltpu.SemaphoreType.DMA,
        ],
    )
    return k(M, s, d, src3, dst3)


def _gru_body(agg2_ref, x_ref, wih_ref, bih_ref, whh_ref, bhh_ref,
              h_ref, stats_ref, sum_ref, sq_ref):
    i = pl.program_id(0)
    agg = agg2_ref[0] + agg2_ref[1]
    xb = x_ref[...]
    gi = lax.dot_general(agg, wih_ref[...], (((1,), (0,)), ((), ())),
                         preferred_element_type=jnp.float32) + bih_ref[...]
    gh = lax.dot_general(xb, whh_ref[...], (((1,), (0,)), ((), ())),
                         preferred_element_type=jnp.float32) + bhh_ref[...]
    r = jax.nn.sigmoid(gi[:, 0:D] + gh[:, 0:D])
    z = jax.nn.sigmoid(gi[:, D:2 * D] + gh[:, D:2 * D])
    n = jnp.tanh(gi[:, 2 * D:] + r * gh[:, 2 * D:])
    h = (1.0 - z) * n + z * xb
    h_ref[...] = h

    @pl.when(i == 0)
    def _init():
        sum_ref[...] = jnp.zeros_like(sum_ref)
        sq_ref[...] = jnp.zeros_like(sq_ref)

    sum_ref[...] += jnp.sum(h, axis=0, keepdims=True)
    sq_ref[...] += jnp.sum(h * h, axis=0, keepdims=True)

    @pl.when(i == GRID - 1)
    def _fin():
        stats_ref[0:1, :] = sum_ref[...]
        stats_ref[1:2, :] = sq_ref[...]


def _gru(agg2, x, WihT, b_ih, WhhT, b_hh):
    return pl.pallas_call(
        _gru_body,
        grid=(GRID,),
        in_specs=[
            pl.BlockSpec((NC, BN, D), lambda i: (0, i, 0)),
            pl.BlockSpec((BN, D), lambda i: (i, 0)),
            pl.BlockSpec((D, 3 * D), lambda i: (0, 0)),
            pl.BlockSpec((1, 3 * D), lambda i: (0, 0)),
            pl.BlockSpec((D, 3 * D), lambda i: (0, 0)),
            pl.BlockSpec((1, 3 * D), lambda i: (0, 0)),
        ],
        out_specs=[
            pl.BlockSpec((BN, D), lambda i: (i, 0)),
            pl.BlockSpec((2, D), lambda i: (0, 0)),
        ],
        out_shape=[
            jax.ShapeDtypeStruct((N, D), jnp.float32),
            jax.ShapeDtypeStruct((2, D), jnp.float32),
        ],
        scratch_shapes=[
            pltpu.VMEM((1, D), jnp.float32),
            pltpu.VMEM((1, D), jnp.float32),
        ],
    )(agg2, x, WihT, b_ih, WhhT, b_hh)


def _norm_body(h_ref, stats_ref, g_ref, b_ref, o_ref):
    nf = jnp.float32(N)
    mean = stats_ref[0:1, :] / nf
    var = stats_ref[1:2, :] / nf - mean * mean
    inv = lax.rsqrt(var + 1e-5)
    o_ref[...] = (h_ref[...] - mean) * (inv * g_ref[...]) + b_ref[...]


def _norm(h, stats, gamma, beta):
    return pl.pallas_call(
        _norm_body,
        grid=(GRID,),
        in_specs=[
            pl.BlockSpec((BN, D), lambda i: (i, 0)),
            pl.BlockSpec((2, D), lambda i: (0, 0)),
            pl.BlockSpec((1, D), lambda i: (0, 0)),
            pl.BlockSpec((1, D), lambda i: (0, 0)),
        ],
        out_specs=pl.BlockSpec((BN, D), lambda i: (i, 0)),
        out_shape=jax.ShapeDtypeStruct((N, D), jnp.float32),
    )(h, stats, gamma.reshape(1, D), beta.reshape(1, D))


def kernel(x, edge_index, W_msg, b_msg, W_att, b_att, W_ih, b_ih, W_hh, b_hh,
           gamma, beta):
    edge_index = edge_index.astype(jnp.int32)
    src3 = edge_index[0].reshape(NW, NWIN, CPW, K)
    dst3 = edge_index[1].reshape(NW, NWIN, CPW, K)

    M, sd = _precompute(x, W_msg, b_msg, W_att, b_att)
    s = sd[:, 0]
    d = sd[:, 1]

    agg2 = _sc_aggregate(M, s, d, src3, dst3)

    h, stats = _gru(agg2, x, W_ih.T, b_ih.reshape(1, 3 * D),
                    W_hh.T, b_hh.reshape(1, 3 * D))
    return _norm(h, stats, gamma, beta)
